# baseline (speedup 1.0000x reference)
import functools

import jax
import jax.numpy as jnp
from jax import lax
from jax.experimental import pallas as pl
from jax.experimental.pallas import tpu as pltpu
from jax.experimental.pallas import tpu_sc as plsc

CHUNK_A = 128
NB = 4


def _make_gather(b, l, d):
    info = plsc.get_sparse_core_info()
    nc, ns = info.num_cores, info.num_subcores
    nw = nc * ns
    rows_per_w = b // nw
    n_groups = rows_per_w // NB
    chunk_b = l - CHUNK_A
    assert rows_per_w * nw == b and n_groups * NB == rows_per_w
    assert n_groups % 2 == 0 and 0 < chunk_b <= 128 and chunk_b % 8 == 0

    mesh = plsc.VectorSubcoreMesh(core_axis_name="c", subcore_axis_name="s")

    @functools.partial(
        pl.kernel,
        mesh=mesh,
        out_type=jax.ShapeDtypeStruct((b, l, d), jnp.float32),
        scratch_types=[
            pltpu.VMEM((rows_per_w, l), jnp.int32),
            pltpu.VMEM((2, NB, l, d), jnp.float32),
            pltpu.SemaphoreType.DMA,
            pltpu.SemaphoreType.DMA,
            pltpu.SemaphoreType.DMA,
            pltpu.SemaphoreType.DMA,
        ],
        compiler_params=pltpu.CompilerParams(use_tc_tiling_on_sc=False),
    )
    def gather_kernel(idx_hbm, table_hbm, out_hbm, idx_v, bufs, g0, g1, s0, s1):
        wid = lax.axis_index("s") * nc + lax.axis_index("c")
        base = wid * rows_per_w
        pltpu.sync_copy(idx_hbm.at[pl.ds(base, rows_per_w)], idx_v)
        gsem = (g0, g1)
        ssem = (s0, s1)

        def issue_gathers(g, h):
            for r in range(NB):
                row = g * NB + r
                pltpu.async_copy(
                    table_hbm.at[idx_v.at[row, pl.ds(0, CHUNK_A)]],
                    bufs.at[h, r, pl.ds(0, CHUNK_A)], gsem[h])
                pltpu.async_copy(
                    table_hbm.at[idx_v.at[row, pl.ds(CHUNK_A, chunk_b)]],
                    bufs.at[h, r, pl.ds(CHUNK_A, chunk_b)], gsem[h])

        def drain_gathers(h):
            for r in range(NB):
                pltpu.make_async_copy(
                    table_hbm.at[idx_v.at[r, pl.ds(0, CHUNK_A)]],
                    bufs.at[h, r, pl.ds(0, CHUNK_A)], gsem[h]).wait()
                pltpu.make_async_copy(
                    table_hbm.at[idx_v.at[r, pl.ds(CHUNK_A, chunk_b)]],
                    bufs.at[h, r, pl.ds(CHUNK_A, chunk_b)], gsem[h]).wait()

        def issue_stores(g, h):
            for r in range(NB):
                pltpu.async_copy(bufs.at[h, r], out_hbm.at[base + g * NB + r],
                                 ssem[h])

        def drain_stores(h):
            for r in range(NB):
                pltpu.make_async_copy(bufs.at[h, r], out_hbm.at[base + r],
                                      ssem[h]).wait()

        issue_gathers(0, 0)

        def body(p, carry):
            gA = 2 * p
            gB = gA + 1
            issue_gathers(gB, 1)
            drain_gathers(0)
            issue_stores(gA, 0)
            drain_gathers(1)
            issue_stores(gB, 1)
            drain_stores(0)

            @pl.when(gA + 2 < n_groups)
            def _():
                issue_gathers(gA + 2, 0)

            drain_stores(1)
            return carry

        lax.fori_loop(0, n_groups // 2, body, 0)

    return gather_kernel


def kernel(input_tensor, weight):
    b, l = input_tensor.shape
    _, d = weight.shape
    gather_kernel = _make_gather(b, l, d)
    return gather_kernel(input_tensor.astype(jnp.int32), weight)
